# Initial kernel scaffold; baseline (speedup 1.0000x reference)
import jax
import jax.numpy as jnp
from jax.experimental import pallas as pl


def kernel(predictions, targets, group_labels):
    raise NotImplementedError("write your pallas kernel here")


# speedup vs baseline: 1.6059x; 1.6059x over previous
import functools

import jax
import jax.numpy as jnp
from jax import lax
from jax.experimental import pallas as pl
from jax.experimental.pallas import tpu as pltpu
from jax.experimental.pallas import tpu_sc as plsc

_G = 8
_ROWS = 16384
_D = 128
_NC = 2
_NS = 16
_NW = _NC * _NS
_RPW = _ROWS // _NW
_CHUNK = 128
_NCHUNK = _RPW // _CHUNK
_PR = 2 * _G + 1


def _sc_body(p_hbm, t_hbm, lab_hbm, out_hbm, pbuf, tbuf, labv, part):
    c = lax.axis_index("c")
    s = lax.axis_index("s")
    wid = s * _NC + c
    base = wid * _RPW

    pltpu.sync_copy(lab_hbm.at[pl.ds(base, _RPW)], labv)

    zero = jnp.zeros((16,), jnp.float32)
    for i in range(_PR):
        part[pl.ds(i * 16, 16)] = zero

    iota = lax.iota(jnp.int32, 16)
    ones = jnp.full((16,), 1.0, jnp.float32)

    def chunk_body(ci, carry):
        rbase = base + ci * _CHUNK
        pltpu.sync_copy(p_hbm.at[pl.ds(rbase, _CHUNK)], pbuf)
        pltpu.sync_copy(t_hbm.at[pl.ds(rbase, _CHUNK)], tbuf)

        def blk_body(bi, carry2):
            r0 = bi * 16
            labvec = labv[pl.ds(ci * _CHUNK + r0, 16)]
            plsc.addupdate_scatter(part, [(labvec + (1 + _G)) * 16 + iota], ones)
            accsq = jnp.zeros((16,), jnp.float32)
            for r in range(16):
                row = r0 + r
                pv = [pbuf[row, pl.ds(k * 16, 16)] for k in range(8)]
                tv = [tbuf[row, pl.ds(k * 16, 16)] for k in range(8)]
                for k in range(8):
                    dd = pv[k] - tv[k]
                    accsq = accsq + dd * dd
                rp = pv[0]
                for k in range(1, 8):
                    rp = rp + pv[k]
                lab_splat = plsc.load_gather(
                    labv, [jnp.full((16,), ci * _CHUNK + row, jnp.int32)])
                plsc.addupdate_scatter(part, [(lab_splat + 1) * 16 + iota], rp)
            part[pl.ds(0, 16)] = part[pl.ds(0, 16)] + accsq
            return carry2

        lax.fori_loop(0, _CHUNK // 16, blk_body, 0)
        return carry

    lax.fori_loop(0, _NCHUNK, chunk_body, 0)
    pltpu.sync_copy(part, out_hbm.at[wid])


@jax.jit
def _sc_partials(predictions, targets, labels):
    mesh = plsc.VectorSubcoreMesh(core_axis_name="c", subcore_axis_name="s")
    f = functools.partial(
        pl.kernel,
        out_type=jax.ShapeDtypeStruct((_NW, _PR * 16), jnp.float32),
        mesh=mesh,
        compiler_params=pltpu.CompilerParams(needs_layout_passes=False),
        scratch_types=[
            pltpu.VMEM((_CHUNK, _D), jnp.float32),
            pltpu.VMEM((_CHUNK, _D), jnp.float32),
            pltpu.VMEM((_RPW,), jnp.int32),
            pltpu.VMEM((_PR * 16,), jnp.float32),
        ],
    )(_sc_body)
    return f(predictions, targets, labels)


def kernel(predictions, targets, group_labels):
    labels = group_labels.astype(jnp.int32)
    parts = _sc_partials(predictions, targets, labels).reshape(_NW, _PR, 16)
    sq = jnp.sum(parts[:, 0, :])
    gs = jnp.sum(parts[:, 1:1 + _G, :], axis=(0, 2))
    cnt = jnp.sum(parts[:, 1 + _G:, :], axis=(0, 2))
    n = predictions.shape[0] * predictions.shape[1]
    base_loss = sq / n
    gm = gs / (cnt * predictions.shape[1])
    mm = jnp.mean(gm)
    penalty = jnp.sum((gm - mm) ** 2) / (_G - 1)
    return base_loss + penalty


# speedup vs baseline: 1.6897x; 1.0522x over previous
import functools

import jax
import jax.numpy as jnp
from jax import lax
from jax.experimental import pallas as pl
from jax.experimental.pallas import tpu as pltpu
from jax.experimental.pallas import tpu_sc as plsc

_G = 8
_ROWS = 16384
_D = 128
_NC = 2
_NS = 16
_NW = _NC * _NS
_RPW = _ROWS // _NW
_CHUNK = 128
_NCHUNK = _RPW // _CHUNK
_PR = 2 * _G + 1

_SPLAT_DNUMS = lax.GatherDimensionNumbers(
    offset_dims=(), collapsed_slice_dims=(0,), start_index_map=(0,))


def _splat(vec, r):
    idx = jnp.full((16, 1), r, jnp.int32)
    return lax.gather(vec, idx, _SPLAT_DNUMS, (1,),
                      mode=lax.GatherScatterMode.PROMISE_IN_BOUNDS)


def _tree8(v):
    a = [v[2 * i] + v[2 * i + 1] for i in range(4)]
    b = [a[0] + a[1], a[2] + a[3]]
    return b[0] + b[1]


def _sc_body(p_hbm, t_hbm, lab_hbm, out_hbm, pbuf, tbuf, labv, part,
             psem, tsem):
    c = lax.axis_index("c")
    s = lax.axis_index("s")
    wid = s * _NC + c
    base = wid * _RPW

    pltpu.sync_copy(lab_hbm.at[pl.ds(base, _RPW)], labv)

    zero = jnp.zeros((16,), jnp.float32)
    for i in range(1, _PR):
        part[pl.ds(i * 16, 16)] = zero

    iota = lax.iota(jnp.int32, 16)
    iota_gs = iota + 16
    iota_cnt = iota + (1 + _G) * 16
    ones = jnp.full((16,), 1.0, jnp.float32)

    def start_chunk(ci):
        rb = base + ci * _CHUNK
        b = ci % 2
        hp = pltpu.async_copy(p_hbm.at[pl.ds(rb, _CHUNK)], pbuf.at[b], psem)
        ht = pltpu.async_copy(t_hbm.at[pl.ds(rb, _CHUNK)], tbuf.at[b], tsem)
        return hp, ht

    handles = start_chunk(0)
    acc = tuple(zero for _ in range(8))

    for ci in range(_NCHUNK):
        b = ci % 2
        handles[0].wait()
        handles[1].wait()
        if ci + 1 < _NCHUNK:
            handles = start_chunk(ci + 1)

        def blk_body(bi, acc_c, _b=b, _ci=ci):
            r0 = bi * 16
            labvec = labv[pl.ds(_ci * _CHUNK + r0, 16)]
            plsc.addupdate_scatter(part, [labvec * 16 + iota_cnt], ones)
            acc_l = list(acc_c)
            for r in range(16):
                row = r0 + r
                pv = [pbuf[_b, row, pl.ds(k * 16, 16)] for k in range(8)]
                tv = [tbuf[_b, row, pl.ds(k * 16, 16)] for k in range(8)]
                dd = [pv[k] - tv[k] for k in range(8)]
                for k in range(8):
                    acc_l[k] = acc_l[k] + dd[k] * dd[k]
                rp = _tree8(pv)
                lab_splat = _splat(labvec, r)
                plsc.addupdate_scatter(part, [lab_splat * 16 + iota_gs], rp)
            return tuple(acc_l)

        acc = lax.fori_loop(0, _CHUNK // 16, blk_body, acc)

    part[pl.ds(0, 16)] = _tree8(acc)
    pltpu.sync_copy(part, out_hbm.at[wid])


@jax.jit
def _sc_partials(predictions, targets, labels):
    mesh = plsc.VectorSubcoreMesh(core_axis_name="c", subcore_axis_name="s")
    f = functools.partial(
        pl.kernel,
        out_type=jax.ShapeDtypeStruct((_NW, _PR * 16), jnp.float32),
        mesh=mesh,
        compiler_params=pltpu.CompilerParams(needs_layout_passes=False),
        scratch_types=[
            pltpu.VMEM((2, _CHUNK, _D), jnp.float32),
            pltpu.VMEM((2, _CHUNK, _D), jnp.float32),
            pltpu.VMEM((_RPW,), jnp.int32),
            pltpu.VMEM((_PR * 16,), jnp.float32),
            pltpu.SemaphoreType.DMA,
            pltpu.SemaphoreType.DMA,
        ],
    )(_sc_body)
    return f(predictions, targets, labels)


def kernel(predictions, targets, group_labels):
    labels = group_labels.astype(jnp.int32)
    parts = _sc_partials(predictions, targets, labels).reshape(_NW, _PR, 16)
    sq = jnp.sum(parts[:, 0, :])
    gs = jnp.sum(parts[:, 1:1 + _G, :], axis=(0, 2))
    cnt = jnp.sum(parts[:, 1 + _G:, :], axis=(0, 2))
    n = predictions.shape[0] * predictions.shape[1]
    base_loss = sq / n
    gm = gs / (cnt * predictions.shape[1])
    mm = jnp.mean(gm)
    penalty = jnp.sum((gm - mm) ** 2) / (_G - 1)
    return base_loss + penalty


# speedup vs baseline: 1.9732x; 1.1678x over previous
import functools

import jax
import jax.numpy as jnp
from jax import lax
from jax.experimental import pallas as pl
from jax.experimental.pallas import tpu as pltpu
from jax.experimental.pallas import tpu_sc as plsc

_G = 8
_ROWS = 16384
_D = 128
_NC = 2
_NS = 16
_NW = _NC * _NS
_RPW = _ROWS // _NW
_CHUNK = 64
_NCHUNK = _RPW // _CHUNK
_PR = 2 * _G + 1
_UNROLL = 8

_SPLAT_DNUMS = lax.GatherDimensionNumbers(
    offset_dims=(), collapsed_slice_dims=(0,), start_index_map=(0,))


def _splat(vec, r):
    idx = jnp.full((16, 1), r, jnp.int32)
    return lax.gather(vec, idx, _SPLAT_DNUMS, (1,),
                      mode=lax.GatherScatterMode.PROMISE_IN_BOUNDS)


def _tree8(v):
    a = [v[2 * i] + v[2 * i + 1] for i in range(4)]
    b = [a[0] + a[1], a[2] + a[3]]
    return b[0] + b[1]


def _sc_body(p_hbm, t_hbm, lab_hbm, out_hbm, pbuf, tbuf, labv, part,
             psem, tsem):
    c = lax.axis_index("c")
    s = lax.axis_index("s")
    wid = s * _NC + c
    base = wid * _RPW

    pltpu.sync_copy(lab_hbm.at[pl.ds(base, _RPW)], labv.at[pl.ds(0, _RPW)])

    zero = jnp.zeros((16,), jnp.float32)
    for i in range(1, _PR):
        part[pl.ds(i * 16, 16)] = zero

    iota = lax.iota(jnp.int32, 16)
    iota_gs = iota + 16
    iota_cnt = iota + (1 + _G) * 16
    ones = jnp.full((16,), 1.0, jnp.float32)

    def start_chunk(ci, b):
        rb = base + ci * _CHUNK
        hp = pltpu.async_copy(p_hbm.at[pl.ds(rb, _CHUNK)], pbuf.at[b], psem)
        ht = pltpu.async_copy(t_hbm.at[pl.ds(rb, _CHUNK)], tbuf.at[b], tsem)
        return hp, ht

    handles = [start_chunk(0, 0), start_chunk(1, 1)]

    def cnt_body(bi, carry):
        labvec = labv[pl.ds(bi * 16, 16)]
        plsc.addupdate_scatter(part, [labvec * 16 + iota_cnt], ones)
        return carry
    lax.fori_loop(0, _RPW // 16, cnt_body, 0)

    def compute_chunk(b, ci, acc_c):
        def blk_body(bi, acc_i, _b=b, _ci=ci):
            r0 = bi * _UNROLL
            labvec = labv[pl.ds(_ci * _CHUNK + r0, 16)]
            acc_l = list(acc_i)
            for r in range(_UNROLL):
                row = r0 + r
                pv = [pbuf[_b, row, pl.ds(k * 16, 16)] for k in range(8)]
                tv = [tbuf[_b, row, pl.ds(k * 16, 16)] for k in range(8)]
                for k in range(8):
                    dd = pv[k] - tv[k]
                    acc_l[k % 4] = acc_l[k % 4] + dd * dd
                rp = _tree8(pv)
                lab_splat = _splat(labvec, r)
                plsc.addupdate_scatter(part, [lab_splat * 16 + iota_gs], rp)
            return tuple(acc_l)
        return lax.fori_loop(0, _CHUNK // _UNROLL, blk_body, acc_c)

    acc = (zero, zero, zero, zero)

    def pair_body(pi, acc_c):
        ci0 = pi * 2
        handles[0][0].wait()
        handles[0][1].wait()
        acc_c = compute_chunk(0, ci0, acc_c)

        @pl.when(ci0 + 2 < _NCHUNK)
        def _():
            start_chunk(ci0 + 2, 0)

        handles[1][0].wait()
        handles[1][1].wait()
        acc_c = compute_chunk(1, ci0 + 1, acc_c)

        @pl.when(ci0 + 3 < _NCHUNK)
        def _():
            start_chunk(ci0 + 3, 1)
        return acc_c

    acc = lax.fori_loop(0, _NCHUNK // 2, pair_body, acc)

    part[pl.ds(0, 16)] = (acc[0] + acc[1]) + (acc[2] + acc[3])
    pltpu.sync_copy(part, out_hbm.at[wid])


@jax.jit
def _sc_partials(predictions, targets, labels):
    mesh = plsc.VectorSubcoreMesh(core_axis_name="c", subcore_axis_name="s")
    f = functools.partial(
        pl.kernel,
        out_type=jax.ShapeDtypeStruct((_NW, _PR * 16), jnp.float32),
        mesh=mesh,
        compiler_params=pltpu.CompilerParams(needs_layout_passes=False),
        scratch_types=[
            pltpu.VMEM((2, _CHUNK, _D), jnp.float32),
            pltpu.VMEM((2, _CHUNK, _D), jnp.float32),
            pltpu.VMEM((_RPW + 16,), jnp.int32),
            pltpu.VMEM((_PR * 16,), jnp.float32),
            pltpu.SemaphoreType.DMA,
            pltpu.SemaphoreType.DMA,
        ],
    )(_sc_body)
    return f(predictions, targets, labels)


def kernel(predictions, targets, group_labels):
    labels = group_labels.astype(jnp.int32)
    parts = _sc_partials(predictions, targets, labels)
    tot = jnp.sum(parts.reshape(_NW, _PR, 16), axis=(0, 2))
    sq = tot[0]
    gs = tot[1:1 + _G]
    cnt = tot[1 + _G:]
    n = predictions.shape[0] * predictions.shape[1]
    base_loss = sq / n
    gm = gs / (cnt * predictions.shape[1])
    mm = jnp.mean(gm)
    penalty = jnp.sum((gm - mm) ** 2) / (_G - 1)
    return base_loss + penalty
